# baseline (speedup 1.0000x reference)
import functools

import numpy as np
import jax
import jax.numpy as jnp
from jax import lax
from jax.experimental import pallas as pl
from jax.experimental.pallas import tpu as pltpu
from jax.experimental.pallas import tpu_sc as plsc

_B, _H, _W, _KP = 4, 256, 256, 16
_NI = _B * _KP
_RH = 64
_NW = 32
_RPW = (_B * _H) // _NW
_NCH = 9
_SAMPLE_IDS = np.repeat(np.arange(_B, dtype=np.int32), _KP)


def _sc_body(ids_hbm, q_hbm, s_hbm, z_hbm, out_hbm,
             ids_v, q_v, s_v, z_v, tab_v, out_v, sem):
    w = lax.axis_index("s") * 2 + lax.axis_index("c")
    row0 = w * _RPW
    b = row0 // _H
    lr0 = row0 - b * _H

    copies = [
        pltpu.async_copy(ids_hbm.at[pl.ds(row0, _RPW)], ids_v, sem),
        pltpu.async_copy(z_hbm.at[pl.ds(row0, _RPW)], z_v, sem),
    ]
    for c in range(4):
        copies.append(pltpu.async_copy(
            q_hbm.at[pl.ds(b * 4 + c, 1), pl.ds(lr0, _RPW)],
            q_v.at[pl.ds(c, 1)], sem))
    for c in range(3):
        copies.append(pltpu.async_copy(
            s_hbm.at[pl.ds(b * 3 + c, 1), pl.ds(lr0, _RPW)],
            s_v.at[pl.ds(c, 1)], sem))

    zero16 = jnp.zeros((16,), jnp.float32)

    @plsc.parallel_loop(0, _NCH * _NI * 16, 16, unroll=8)
    def _zero_body(i):
        tab_v[pl.ds(i, 16)] = zero16

    for cp in copies:
        cp.wait()

    lanes = lax.iota(jnp.int32, 16)
    ones16 = jnp.ones((16,), jnp.float32)

    @plsc.parallel_loop(0, _RPW * _W, 16, unroll=4)
    def _vec_body(i):
        r = jax.lax.shift_right_logical(i, 8)
        off = jax.lax.bitwise_and(i, _W - 1)
        bin16 = ids_v[r, pl.ds(off, 16)] - 1
        idx0 = bin16 * 16 + lanes
        for c in range(4):
            v = q_v[c, r, pl.ds(off, 16)]
            plsc.addupdate_scatter(tab_v, [idx0 + c * (_NI * 16)], v)
        for c in range(3):
            v = s_v[c, r, pl.ds(off, 16)]
            plsc.addupdate_scatter(tab_v, [idx0 + (4 + c) * (_NI * 16)], v)
        v = z_v[r, pl.ds(off, 16)]
        plsc.addupdate_scatter(tab_v, [idx0 + 7 * (_NI * 16)], v)
        plsc.addupdate_scatter(tab_v, [idx0 + 8 * (_NI * 16)], ones16)

    lanes16 = lanes * 16
    for g in range(_NCH * _NI // 16):
        acc = plsc.load_gather(tab_v, [lanes16 + g * 256])
        for k in range(1, 16):
            acc = acc + plsc.load_gather(tab_v, [lanes16 + (g * 256 + k)])
        out_v[pl.ds(g * 16, 16)] = acc

    pltpu.sync_copy(out_v, out_hbm.at[w])


_sc_sums = functools.partial(
    pl.kernel,
    out_type=jax.ShapeDtypeStruct((_NW, _NCH * _NI), jnp.float32),
    mesh=plsc.VectorSubcoreMesh(core_axis_name="c", subcore_axis_name="s"),
    compiler_params=pltpu.CompilerParams(needs_layout_passes=False),
    scratch_types=[
        pltpu.VMEM((_RPW, _W), jnp.int32),
        pltpu.VMEM((4, _RPW, _W), jnp.float32),
        pltpu.VMEM((3, _RPW, _W), jnp.float32),
        pltpu.VMEM((_RPW, _W), jnp.float32),
        pltpu.VMEM((_NCH * _NI * 16,), jnp.float32),
        pltpu.VMEM((_NCH * _NI,), jnp.float32),
        pltpu.SemaphoreType.DMA,
    ],
)(_sc_body)


def _dense_body(ids_ref, mask_ref, xy_ref, imask_ref, xyout_ref, cls_ref,
                accm_ref):
    b = pl.program_id(0)
    r = pl.program_id(1)
    nr = pl.num_programs(1)

    @pl.when(r == 0)
    def _init():
        accm_ref[...] = jnp.zeros_like(accm_ref)

    ids = ids_ref[0]
    mcls = mask_ref[0]
    xy0 = xy_ref[0, 0]
    xy1 = xy_ref[0, 1]
    base = b * _KP + 1
    for j in range(_KP):
        bm = ids == (base + j)
        bmf = bm.astype(jnp.float32)
        imask_ref[j] = bmf
        xyout_ref[j, 0] = bmf * xy0
        xyout_ref[j, 1] = bmf * xy1
        cm = jnp.max(jnp.where(bm, mcls, 0), axis=0)
        accm_ref[j] = jnp.maximum(accm_ref[j], cm)

    @pl.when(r == nr - 1)
    def _fin():
        cls = jnp.max(accm_ref[...], axis=-1, keepdims=True)
        cls_ref[...] = jnp.broadcast_to(cls, (_KP, 128))


def _dense(instance_ids, mask, xy):
    grid = (_B, _H // _RH)
    out_shapes = (
        jax.ShapeDtypeStruct((_NI, _H, _W), jnp.float32),
        jax.ShapeDtypeStruct((_NI, 2, _H, _W), jnp.float32),
        jax.ShapeDtypeStruct((_NI, 128), jnp.int32),
    )
    return pl.pallas_call(
        _dense_body,
        grid=grid,
        in_specs=[
            pl.BlockSpec((1, _RH, _W), lambda b, r: (b, r, 0)),
            pl.BlockSpec((1, _RH, _W), lambda b, r: (b, r, 0)),
            pl.BlockSpec((1, 2, _RH, _W), lambda b, r: (b, 0, r, 0)),
        ],
        out_specs=[
            pl.BlockSpec((_KP, _RH, _W), lambda b, r: (b, r, 0)),
            pl.BlockSpec((_KP, 2, _RH, _W), lambda b, r: (b, 0, r, 0)),
            pl.BlockSpec((_KP, 128), lambda b, r: (b, 0)),
        ],
        out_shape=out_shapes,
        scratch_shapes=[
            pltpu.VMEM((_KP, _W), jnp.int32),
        ],
    )(instance_ids, mask, xy)


def _fin_body(p_ref, clso_ref, qn_ref, sm_ref, ze_ref, cls_ref):
    tot = jnp.sum(p_ref[...], axis=0)
    ch = [tot[c * _NI:(c + 1) * _NI] for c in range(_NCH)]
    cnt = ch[8]
    qm = [ch[c] / cnt for c in range(4)]
    nrm = jnp.sqrt(qm[0] * qm[0] + qm[1] * qm[1] + qm[2] * qm[2]
                   + qm[3] * qm[3])
    rows = ([qm[c] / nrm for c in range(4)]
            + [ch[4 + c] / cnt for c in range(3)]
            + [jnp.exp(ch[7] / cnt)])
    m = jnp.concatenate([r[None, :] for r in rows], axis=0)
    t = m.T
    qn_ref[...] = t[:, 0:4]
    sm_ref[...] = t[:, 4:7]
    ze_ref[...] = t[:, 7:8]
    cls_ref[...] = clso_ref[:, 0:1]


def _finalize(part, clso):
    return pl.pallas_call(
        _fin_body,
        out_shape=(
            jax.ShapeDtypeStruct((_NI, 4), jnp.float32),
            jax.ShapeDtypeStruct((_NI, 3), jnp.float32),
            jax.ShapeDtypeStruct((_NI, 1), jnp.float32),
            jax.ShapeDtypeStruct((_NI, 1), jnp.int32),
        ),
    )(part, clso)


@jax.jit
def kernel(mask, instance_ids, quaternion, scales, xy, z):
    ids_r = instance_ids.reshape(_B * _H, _W)
    q_r = quaternion.reshape(_B * 4, _H, _W)
    s_r = scales.reshape(_B * 3, _H, _W)
    z_r = z.reshape(_B * _H, _W)

    part = _sc_sums(ids_r, q_r, s_r, z_r)
    imask, xyout, clso = _dense(instance_ids, mask, xy)
    qn, sm, ze, cls2 = _finalize(part, clso)

    cls = cls2.reshape(_NI)
    sample_ids = jnp.asarray(_SAMPLE_IDS)
    return (cls, imask, sample_ids, qn, sm, xyout, ze)


# speedup vs baseline: 1.3917x; 1.3917x over previous
import jax
import jax.numpy as jnp
from jax.experimental import pallas as pl
from jax.experimental.pallas import tpu as pltpu

_B, _H, _W, _KP = 4, 256, 256, 16
_RH = 64


def _agg_body(ids_ref, mask_ref, q_ref, s_ref, xy_ref, z_ref,
              imask_ref, xyout_ref, stats_ref, acc_ref, accm_ref):
    b = pl.program_id(0)
    r = pl.program_id(1)
    nr = pl.num_programs(1)

    @pl.when(r == 0)
    def _init():
        acc_ref[...] = jnp.zeros_like(acc_ref)
        accm_ref[...] = jnp.zeros_like(accm_ref)

    ids = ids_ref[0]
    mcls = mask_ref[0]
    xy0 = xy_ref[0, 0]
    xy1 = xy_ref[0, 1]
    chans = ([q_ref[0, c] for c in range(4)]
             + [s_ref[0, c] for c in range(3)]
             + [z_ref[0]])
    base = b * _KP + 1
    bmfs = []
    for j in range(_KP):
        bm = ids == (base + j)
        bmf = bm.astype(jnp.float32)
        bmfs.append(bmf[None])
        imask_ref[j] = bmf
        xyout_ref[j, 0] = bmf * xy0
        xyout_ref[j, 1] = bmf * xy1
        cm = jnp.max(jnp.where(bm, mcls, 0), axis=0)
        accm_ref[j] = jnp.maximum(accm_ref[j], cm)

    bmat = jnp.concatenate(bmfs, axis=0)
    cmat = jnp.concatenate(
        [c[None] for c in chans] + [jnp.ones((1, _RH, _W), jnp.float32)],
        axis=0)
    acc_ref[...] += jax.lax.dot_general(
        bmat.reshape(_KP, _RH * _W), cmat.reshape(9, _RH * _W),
        (((1,), (1,)), ((), ())),
        preferred_element_type=jnp.float32)

    @pl.when(r == nr - 1)
    def _fin():
        red = acc_ref[...]
        cnt = red[:, 8:9]
        qm = red[:, 0:4] / cnt
        sm = red[:, 4:7] / cnt
        zm = red[:, 7:8] / cnt
        qn = qm / jnp.sqrt(jnp.sum(qm * qm, axis=1, keepdims=True))
        ze = jnp.exp(zm)
        cls = jnp.max(accm_ref[...], axis=-1).astype(jnp.float32)[:, None]
        out9 = jnp.concatenate([qn, sm, ze, cls], axis=1)
        stats_ref[...] = jnp.concatenate(
            [out9, jnp.zeros((_KP, 128 - 9), jnp.float32)], axis=1)


def _run(mask, instance_ids, quaternion, scales, xy, z, interpret=False):
    grid = (_B, _H // _RH)
    out_shapes = (
        jax.ShapeDtypeStruct((_B * _KP, _H, _W), jnp.float32),
        jax.ShapeDtypeStruct((_B * _KP, 2, _H, _W), jnp.float32),
        jax.ShapeDtypeStruct((_B * _KP, 128), jnp.float32),
    )
    return pl.pallas_call(
        _agg_body,
        grid=grid,
        in_specs=[
            pl.BlockSpec((1, _RH, _W), lambda b, r: (b, r, 0)),
            pl.BlockSpec((1, _RH, _W), lambda b, r: (b, r, 0)),
            pl.BlockSpec((1, 4, _RH, _W), lambda b, r: (b, 0, r, 0)),
            pl.BlockSpec((1, 3, _RH, _W), lambda b, r: (b, 0, r, 0)),
            pl.BlockSpec((1, 2, _RH, _W), lambda b, r: (b, 0, r, 0)),
            pl.BlockSpec((1, _RH, _W), lambda b, r: (b, r, 0)),
        ],
        out_specs=[
            pl.BlockSpec((_KP, _RH, _W), lambda b, r: (b, r, 0)),
            pl.BlockSpec((_KP, 2, _RH, _W), lambda b, r: (b, 0, r, 0)),
            pl.BlockSpec((_KP, 128), lambda b, r: (b, 0)),
        ],
        out_shape=out_shapes,
        scratch_shapes=[
            pltpu.VMEM((_KP, 9), jnp.float32),
            pltpu.VMEM((_KP, _W), jnp.int32),
        ],
        interpret=interpret,
    )(instance_ids, mask, quaternion, scales, xy, z)


@jax.jit
def kernel(mask, instance_ids, quaternion, scales, xy, z):
    imask, xyout, stats = _run(mask, instance_ids, quaternion, scales, xy, z)
    cls = stats[:, 8].astype(jnp.int32)
    qn = stats[:, 0:4]
    sm = stats[:, 4:7]
    ze = stats[:, 7:8]
    sample_ids = jnp.repeat(jnp.arange(_B, dtype=jnp.int32), _KP)
    return (cls, imask, sample_ids, qn, sm, xyout, ze)


# speedup vs baseline: 1.5756x; 1.1322x over previous
import jax
import jax.numpy as jnp
from jax.experimental import pallas as pl
from jax.experimental.pallas import tpu as pltpu

_B, _H, _W, _KP = 4, 256, 256, 16
_RH = 128


def _agg_body(ids_ref, mask_ref, q_ref, s_ref, xy_ref, z_ref,
              imask_ref, xyout_ref, stats_ref, acc_ref, accm_ref):
    b = pl.program_id(0)
    r = pl.program_id(1)
    nr = pl.num_programs(1)

    @pl.when(r == 0)
    def _init():
        acc_ref[...] = jnp.zeros_like(acc_ref)
        accm_ref[...] = jnp.zeros_like(accm_ref)

    ids = ids_ref[0]
    mcls = mask_ref[0]
    xy0 = xy_ref[0, 0]
    xy1 = xy_ref[0, 1]
    chans = ([q_ref[0, c] for c in range(4)]
             + [s_ref[0, c] for c in range(3)]
             + [z_ref[0]])
    base = b * _KP + 1
    bmfs = []
    for j in range(_KP):
        bm = ids == (base + j)
        bmf = bm.astype(jnp.float32)
        bmfs.append(bmf[None])
        imask_ref[j] = bmf
        xyout_ref[j, 0] = bmf * xy0
        xyout_ref[j, 1] = bmf * xy1
        cm = jnp.max(jnp.where(bm, mcls, 0), axis=0)
        accm_ref[j] = jnp.maximum(accm_ref[j], cm)

    bmat = jnp.concatenate(bmfs, axis=0)
    cmat = jnp.concatenate(
        [c[None] for c in chans] + [jnp.ones((1, _RH, _W), jnp.float32)],
        axis=0)
    acc_ref[...] += jax.lax.dot_general(
        bmat.reshape(_KP, _RH * _W), cmat.reshape(9, _RH * _W),
        (((1,), (1,)), ((), ())),
        preferred_element_type=jnp.float32)

    @pl.when(r == nr - 1)
    def _fin():
        red = acc_ref[...]
        cnt = red[:, 8:9]
        qm = red[:, 0:4] / cnt
        sm = red[:, 4:7] / cnt
        zm = red[:, 7:8] / cnt
        qn = qm / jnp.sqrt(jnp.sum(qm * qm, axis=1, keepdims=True))
        ze = jnp.exp(zm)
        cls = jnp.max(accm_ref[...], axis=-1).astype(jnp.float32)[:, None]
        out9 = jnp.concatenate([qn, sm, ze, cls], axis=1)
        stats_ref[...] = jnp.concatenate(
            [out9, jnp.zeros((_KP, 128 - 9), jnp.float32)], axis=1)


def _run(mask, instance_ids, quaternion, scales, xy, z, interpret=False):
    grid = (_B, _H // _RH)
    out_shapes = (
        jax.ShapeDtypeStruct((_B * _KP, _H, _W), jnp.float32),
        jax.ShapeDtypeStruct((_B * _KP, 2, _H, _W), jnp.float32),
        jax.ShapeDtypeStruct((_B * _KP, 128), jnp.float32),
    )
    return pl.pallas_call(
        _agg_body,
        grid=grid,
        in_specs=[
            pl.BlockSpec((1, _RH, _W), lambda b, r: (b, r, 0)),
            pl.BlockSpec((1, _RH, _W), lambda b, r: (b, r, 0)),
            pl.BlockSpec((1, 4, _RH, _W), lambda b, r: (b, 0, r, 0)),
            pl.BlockSpec((1, 3, _RH, _W), lambda b, r: (b, 0, r, 0)),
            pl.BlockSpec((1, 2, _RH, _W), lambda b, r: (b, 0, r, 0)),
            pl.BlockSpec((1, _RH, _W), lambda b, r: (b, r, 0)),
        ],
        out_specs=[
            pl.BlockSpec((_KP, _RH, _W), lambda b, r: (b, r, 0)),
            pl.BlockSpec((_KP, 2, _RH, _W), lambda b, r: (b, 0, r, 0)),
            pl.BlockSpec((_KP, 128), lambda b, r: (b, 0)),
        ],
        out_shape=out_shapes,
        scratch_shapes=[
            pltpu.VMEM((_KP, 9), jnp.float32),
            pltpu.VMEM((_KP, _W), jnp.int32),
        ],
        interpret=interpret,
    )(instance_ids, mask, quaternion, scales, xy, z)


@jax.jit
def kernel(mask, instance_ids, quaternion, scales, xy, z):
    imask, xyout, stats = _run(mask, instance_ids, quaternion, scales, xy, z)
    cls = stats[:, 8].astype(jnp.int32)
    qn = stats[:, 0:4]
    sm = stats[:, 4:7]
    ze = stats[:, 7:8]
    sample_ids = jnp.repeat(jnp.arange(_B, dtype=jnp.int32), _KP)
    return (cls, imask, sample_ids, qn, sm, xyout, ze)


# speedup vs baseline: 1.6295x; 1.0342x over previous
import jax
import jax.numpy as jnp
from jax.experimental import pallas as pl
from jax.experimental.pallas import tpu as pltpu

_B, _H, _W, _KP = 4, 256, 256, 16
_RH = 256


def _agg_body(ids_ref, mask_ref, q_ref, s_ref, xy_ref, z_ref,
              imask_ref, xyout_ref, stats_ref, acc_ref, accm_ref):
    b = pl.program_id(0)
    r = pl.program_id(1)
    nr = pl.num_programs(1)

    @pl.when(r == 0)
    def _init():
        acc_ref[...] = jnp.zeros_like(acc_ref)
        accm_ref[...] = jnp.zeros_like(accm_ref)

    ids = ids_ref[0]
    mcls = mask_ref[0]
    xy0 = xy_ref[0, 0]
    xy1 = xy_ref[0, 1]
    chans = ([q_ref[0, c] for c in range(4)]
             + [s_ref[0, c] for c in range(3)]
             + [z_ref[0]])
    base = b * _KP + 1
    bmfs = []
    for j in range(_KP):
        bm = ids == (base + j)
        bmf = bm.astype(jnp.float32)
        bmfs.append(bmf[None])
        imask_ref[j] = bmf
        xyout_ref[j, 0] = bmf * xy0
        xyout_ref[j, 1] = bmf * xy1
        cm = jnp.max(jnp.where(bm, mcls, 0), axis=0)
        accm_ref[j] = jnp.maximum(accm_ref[j], cm)

    bmat = jnp.concatenate(bmfs, axis=0)
    cmat = jnp.concatenate(
        [c[None] for c in chans] + [jnp.ones((1, _RH, _W), jnp.float32)],
        axis=0)
    acc_ref[...] += jax.lax.dot_general(
        bmat.reshape(_KP, _RH * _W), cmat.reshape(9, _RH * _W),
        (((1,), (1,)), ((), ())),
        preferred_element_type=jnp.float32)

    @pl.when(r == nr - 1)
    def _fin():
        red = acc_ref[...]
        cnt = red[:, 8:9]
        qm = red[:, 0:4] / cnt
        sm = red[:, 4:7] / cnt
        zm = red[:, 7:8] / cnt
        qn = qm / jnp.sqrt(jnp.sum(qm * qm, axis=1, keepdims=True))
        ze = jnp.exp(zm)
        cls = jnp.max(accm_ref[...], axis=-1).astype(jnp.float32)[:, None]
        out9 = jnp.concatenate([qn, sm, ze, cls], axis=1)
        stats_ref[...] = jnp.concatenate(
            [out9, jnp.zeros((_KP, 128 - 9), jnp.float32)], axis=1)


def _run(mask, instance_ids, quaternion, scales, xy, z, interpret=False):
    grid = (_B, _H // _RH)
    out_shapes = (
        jax.ShapeDtypeStruct((_B * _KP, _H, _W), jnp.float32),
        jax.ShapeDtypeStruct((_B * _KP, 2, _H, _W), jnp.float32),
        jax.ShapeDtypeStruct((_B * _KP, 128), jnp.float32),
    )
    return pl.pallas_call(
        _agg_body,
        grid=grid,
        in_specs=[
            pl.BlockSpec((1, _RH, _W), lambda b, r: (b, r, 0)),
            pl.BlockSpec((1, _RH, _W), lambda b, r: (b, r, 0)),
            pl.BlockSpec((1, 4, _RH, _W), lambda b, r: (b, 0, r, 0)),
            pl.BlockSpec((1, 3, _RH, _W), lambda b, r: (b, 0, r, 0)),
            pl.BlockSpec((1, 2, _RH, _W), lambda b, r: (b, 0, r, 0)),
            pl.BlockSpec((1, _RH, _W), lambda b, r: (b, r, 0)),
        ],
        out_specs=[
            pl.BlockSpec((_KP, _RH, _W), lambda b, r: (b, r, 0)),
            pl.BlockSpec((_KP, 2, _RH, _W), lambda b, r: (b, 0, r, 0)),
            pl.BlockSpec((_KP, 128), lambda b, r: (b, 0)),
        ],
        out_shape=out_shapes,
        scratch_shapes=[
            pltpu.VMEM((_KP, 9), jnp.float32),
            pltpu.VMEM((_KP, _W), jnp.int32),
        ],
        interpret=interpret,
    )(instance_ids, mask, quaternion, scales, xy, z)


@jax.jit
def kernel(mask, instance_ids, quaternion, scales, xy, z):
    imask, xyout, stats = _run(mask, instance_ids, quaternion, scales, xy, z)
    cls = stats[:, 8].astype(jnp.int32)
    qn = stats[:, 0:4]
    sm = stats[:, 4:7]
    ze = stats[:, 7:8]
    sample_ids = jnp.repeat(jnp.arange(_B, dtype=jnp.int32), _KP)
    return (cls, imask, sample_ids, qn, sm, xyout, ze)


# speedup vs baseline: 1.7108x; 1.0499x over previous
import numpy as np
import jax
import jax.numpy as jnp
from jax.experimental import pallas as pl
from jax.experimental.pallas import tpu as pltpu

_B, _H, _W, _KP = 4, 256, 256, 16
_RH = 256
_SAMPLE_IDS = np.repeat(np.arange(_B, dtype=np.int32), _KP)


def _agg_body(ids_ref, mask_ref, q_ref, s_ref, xy_ref, z_ref,
              imask_ref, xyout_ref, qn_ref, sm_ref, ze_ref, cls_ref,
              acc_ref, accm_ref):
    b = pl.program_id(0)
    r = pl.program_id(1)
    nr = pl.num_programs(1)

    @pl.when(r == 0)
    def _init():
        acc_ref[...] = jnp.zeros_like(acc_ref)
        accm_ref[...] = jnp.zeros_like(accm_ref)

    ids = ids_ref[0]
    mcls = mask_ref[0]
    xy0 = xy_ref[0, 0]
    xy1 = xy_ref[0, 1]
    chans = ([q_ref[0, c] for c in range(4)]
             + [s_ref[0, c] for c in range(3)]
             + [z_ref[0]])
    base = b * _KP + 1
    bmfs = []
    for j in range(_KP):
        bm = ids == (base + j)
        bmf = bm.astype(jnp.float32)
        bmfs.append(bmf[None])
        imask_ref[j] = bmf
        xyout_ref[j, 0] = bmf * xy0
        xyout_ref[j, 1] = bmf * xy1
        cm = jnp.max(jnp.where(bm, mcls, 0), axis=0)
        accm_ref[j] = jnp.maximum(accm_ref[j], cm)

    bmat = jnp.concatenate(bmfs, axis=0)
    cmat = jnp.concatenate(
        [c[None] for c in chans] + [jnp.ones((1, _RH, _W), jnp.float32)],
        axis=0)
    acc_ref[...] += jax.lax.dot_general(
        bmat.reshape(_KP, _RH * _W), cmat.reshape(9, _RH * _W),
        (((1,), (1,)), ((), ())),
        preferred_element_type=jnp.float32)

    @pl.when(r == nr - 1)
    def _fin():
        red = acc_ref[...]
        cnt = red[:, 8:9]
        qm = red[:, 0:4] / cnt
        sm = red[:, 4:7] / cnt
        zm = red[:, 7:8] / cnt
        qn_ref[...] = qm / jnp.sqrt(jnp.sum(qm * qm, axis=1, keepdims=True))
        sm_ref[...] = sm
        ze_ref[...] = jnp.exp(zm)
        cls_ref[...] = jnp.max(accm_ref[...], axis=-1)[:, None]


def _run(mask, instance_ids, quaternion, scales, xy, z, interpret=False):
    grid = (_B, _H // _RH)
    out_shapes = (
        jax.ShapeDtypeStruct((_B * _KP, _H, _W), jnp.float32),
        jax.ShapeDtypeStruct((_B * _KP, 2, _H, _W), jnp.float32),
        jax.ShapeDtypeStruct((_B * _KP, 4), jnp.float32),
        jax.ShapeDtypeStruct((_B * _KP, 3), jnp.float32),
        jax.ShapeDtypeStruct((_B * _KP, 1), jnp.float32),
        jax.ShapeDtypeStruct((_B * _KP, 1), jnp.int32),
    )
    return pl.pallas_call(
        _agg_body,
        grid=grid,
        in_specs=[
            pl.BlockSpec((1, _RH, _W), lambda b, r: (b, r, 0)),
            pl.BlockSpec((1, _RH, _W), lambda b, r: (b, r, 0)),
            pl.BlockSpec((1, 4, _RH, _W), lambda b, r: (b, 0, r, 0)),
            pl.BlockSpec((1, 3, _RH, _W), lambda b, r: (b, 0, r, 0)),
            pl.BlockSpec((1, 2, _RH, _W), lambda b, r: (b, 0, r, 0)),
            pl.BlockSpec((1, _RH, _W), lambda b, r: (b, r, 0)),
        ],
        out_specs=[
            pl.BlockSpec((_KP, _RH, _W), lambda b, r: (b, r, 0)),
            pl.BlockSpec((_KP, 2, _RH, _W), lambda b, r: (b, 0, r, 0)),
            pl.BlockSpec((_KP, 4), lambda b, r: (b, 0)),
            pl.BlockSpec((_KP, 3), lambda b, r: (b, 0)),
            pl.BlockSpec((_KP, 1), lambda b, r: (b, 0)),
            pl.BlockSpec((_KP, 1), lambda b, r: (b, 0)),
        ],
        out_shape=out_shapes,
        scratch_shapes=[
            pltpu.VMEM((_KP, 9), jnp.float32),
            pltpu.VMEM((_KP, _W), jnp.int32),
        ],
        interpret=interpret,
    )(instance_ids, mask, quaternion, scales, xy, z)


@jax.jit
def kernel(mask, instance_ids, quaternion, scales, xy, z):
    imask, xyout, qn, sm, ze, cls2 = _run(
        mask, instance_ids, quaternion, scales, xy, z)
    cls = cls2.reshape(_B * _KP)
    sample_ids = jnp.asarray(_SAMPLE_IDS)
    return (cls, imask, sample_ids, qn, sm, xyout, ze)
